# Initial kernel scaffold; baseline (speedup 1.0000x reference)
import jax
import jax.numpy as jnp
from jax.experimental import pallas as pl


def kernel(flatten, features, pfcnetworks, mdcbcnetworks, v1cnetworks, shanetworks, pfcnetworkweights, mdcbcnetworkweights, v1cnetworkweights, shanetworkweights, W1, b1, gamma, beta, W2, b2, Wg, bg):
    raise NotImplementedError("write your pallas kernel here")


# speedup vs baseline: 124.3931x; 124.3931x over previous
import functools

import jax
import jax.numpy as jnp
from jax import lax
from jax.experimental import pallas as pl
from jax.experimental.pallas import tpu as pltpu
from jax.experimental.pallas import tpu_sc as plsc

N_GENES = 50000
UNIT = 15
FEAT = 20
H1 = 4
E_EDGES = 3200000
NEXP = 4

NC, NS = 2, 16
NPAD = 50176
NT = NPAD // NS
NB = 512
NBLK = NPAD // NB

RW = 8
E_SC = E_EDGES // NC
CH_E = 1600
NCHUNK = E_SC // CH_E
CPT = (NCHUNK + NS - 1) // NS

@functools.cache
def _mesh():
    return plsc.VectorSubcoreMesh(core_axis_name="c", subcore_axis_name="s",
                                  num_cores=NC, num_subcores=NS)


def _sc_deg(d0, d1, d2, d3, ones_src, zeros1):
    def body(d0r, d1r, d2r, d3r, ones_hbm, zer_hbm, out,
             deg_sh, idx_v, ones_v):
        cid = lax.axis_index("c")
        sid = lax.axis_index("s")
        pltpu.sync_copy(ones_hbm, ones_v)
        for k, dref in enumerate((d0r, d1r, d2r, d3r)):
            pltpu.sync_copy(zer_hbm, deg_sh.at[pl.ds(sid * NT, NT)])
            plsc.subcore_barrier()

            def chunk(j, carry):
                c = sid + NS * j
                @pl.when(c < NCHUNK)
                def _():
                    base = cid * E_SC + c * CH_E
                    pltpu.sync_copy(dref.at[pl.ds(base, CH_E)], idx_v)
                    pltpu.sync_copy(ones_v, deg_sh.at[idx_v], add=True)
                return carry

            lax.fori_loop(0, CPT, chunk, 0)
            plsc.subcore_barrier()
            pltpu.sync_copy(deg_sh.at[pl.ds(sid * NT, NT)],
                            out.at[cid, k, pl.ds(sid * NT, NT)])
            plsc.subcore_barrier()

    fn = pl.kernel(
        body,
        out_type=jax.ShapeDtypeStruct((NC, NEXP, NPAD, RW), jnp.float32),
        mesh=_mesh(),
        compiler_params=pltpu.CompilerParams(use_tc_tiling_on_sc=False),
        scratch_types=[
            pltpu.VMEM_SHARED((NPAD, RW), jnp.float32),
            pltpu.VMEM((CH_E,), jnp.int32),
            pltpu.VMEM((CH_E, RW), jnp.float32),
        ],
    )
    return fn(d0, d1, d2, d3, ones_src, zeros1)


def _sc_agg(srcs, dsts, p, zerosw):
    def body(s0, s1, s2, s3, d0, d1, d2, d3, p_hbm, zer_hbm, out,
             p_sh, acc_sh, idx_s, idx_d, rows_v, stage_v):
        cid = lax.axis_index("c")
        sid = lax.axis_index("s")
        srefs = (s0, s1, s2, s3)
        drefs = (d0, d1, d2, d3)
        for k in range(NEXP):
            sl = pl.ds(sid * NT, NT)
            pltpu.sync_copy(p_hbm.at[k, sl, :], stage_v)
            pltpu.sync_copy(stage_v, p_sh.at[sl, :])
            pltpu.sync_copy(zer_hbm, acc_sh.at[sl, :])
            plsc.subcore_barrier()

            def chunk(j, carry):
                c = sid + NS * j
                @pl.when(c < NCHUNK)
                def _():
                    base = cid * E_SC + c * CH_E
                    pltpu.sync_copy(srefs[k].at[pl.ds(base, CH_E)], idx_s)
                    pltpu.sync_copy(drefs[k].at[pl.ds(base, CH_E)], idx_d)
                    pltpu.sync_copy(p_sh.at[idx_s], rows_v)
                    pltpu.sync_copy(rows_v, acc_sh.at[idx_d], add=True)
                return carry

            lax.fori_loop(0, CPT, chunk, 0)
            plsc.subcore_barrier()
            pltpu.sync_copy(acc_sh.at[sl, :], out.at[cid, k, sl, :])
            plsc.subcore_barrier()

    fn = pl.kernel(
        body,
        out_type=jax.ShapeDtypeStruct((NC, NEXP, NPAD, RW), jnp.float32),
        mesh=_mesh(),
        compiler_params=pltpu.CompilerParams(use_tc_tiling_on_sc=False),
        scratch_types=[
            pltpu.VMEM_SHARED((NPAD, RW), jnp.float32),
            pltpu.VMEM_SHARED((NPAD, RW), jnp.float32),
            pltpu.VMEM((CH_E,), jnp.int32),
            pltpu.VMEM((CH_E,), jnp.int32),
            pltpu.VMEM((CH_E, RW), jnp.float32),
            pltpu.VMEM((NT, RW), jnp.float32),
        ],
    )
    return fn(*srcs, *dsts, p, zerosw)


def _tc1_body(deg_ref, xT_ref, w1_ref, dis_ref, p1_ref):
    deg = deg_ref[0, :, :, 0] + deg_ref[1, :, :, 0]
    dis = lax.rsqrt(deg + 2.0)
    dis_ref[...] = dis
    xb = xT_ref[...]
    zero = jnp.zeros((NB,), jnp.float32)
    for k in range(NEXP):
        cols = []
        for f in range(H1):
            acc = w1_ref[k, 0, f] * xb[0, :]
            for j in range(1, UNIT):
                acc = acc + w1_ref[k, j, f] * xb[j, :]
            cols.append(acc * dis[k, :])
        cols += [zero] * (RW - H1)
        p1_ref[k, :, :] = jnp.stack(cols, axis=-1)


def _tc1(deg, xT, W1):
    return pl.pallas_call(
        _tc1_body,
        grid=(NBLK,),
        in_specs=[
            pl.BlockSpec((NC, NEXP, NB, RW), lambda i: (0, 0, i, 0)),
            pl.BlockSpec((UNIT, NB), lambda i: (0, i)),
            pl.BlockSpec((NEXP, UNIT, H1), lambda i: (0, 0, 0)),
        ],
        out_specs=[
            pl.BlockSpec((NEXP, NB), lambda i: (0, i)),
            pl.BlockSpec((NEXP, NB, RW), lambda i: (0, i, 0)),
        ],
        out_shape=[
            jax.ShapeDtypeStruct((NEXP, NPAD), jnp.float32),
            jax.ShapeDtypeStruct((NEXP, NPAD, RW), jnp.float32),
        ],
    )(deg, xT, W1)


def _tc2a_body(acc_ref, p1_ref, dis_ref, b1_ref, r_ref, s1_ref, s2_ref):
    i = pl.program_id(0)

    @pl.when(i == 0)
    def _():
        s1_ref[...] = jnp.zeros((NEXP, H1), jnp.float32)
        s2_ref[...] = jnp.zeros((NEXP, H1), jnp.float32)

    node = i * NB + lax.broadcasted_iota(jnp.int32, (NB, H1), 0)
    valid = (node < N_GENES).astype(jnp.float32)
    for k in range(NEXP):
        a = acc_ref[0, k, :, :H1] + acc_ref[1, k, :, :H1]
        p1 = p1_ref[k, :, :H1]
        dis = dis_ref[k, :][:, None]
        out1 = dis * (a + 2.0 * p1) + b1_ref[k, :][None, :]
        r = jnp.maximum(out1, 0.0) * valid
        r_ref[k] = r
        s1_ref[k, :] += jnp.sum(r, axis=0)
        s2_ref[k, :] += jnp.sum(r * r, axis=0)


def _tc2a(acc1, p1, dis, b1):
    return pl.pallas_call(
        _tc2a_body,
        grid=(NBLK,),
        in_specs=[
            pl.BlockSpec((NC, NEXP, NB, RW), lambda i: (0, 0, i, 0)),
            pl.BlockSpec((NEXP, NB, RW), lambda i: (0, i, 0)),
            pl.BlockSpec((NEXP, NB), lambda i: (0, i)),
            pl.BlockSpec((NEXP, H1), lambda i: (0, 0)),
        ],
        out_specs=[
            pl.BlockSpec((NEXP, NB, H1), lambda i: (0, i, 0)),
            pl.BlockSpec((NEXP, H1), lambda i: (0, 0)),
            pl.BlockSpec((NEXP, H1), lambda i: (0, 0)),
        ],
        out_shape=[
            jax.ShapeDtypeStruct((NEXP, NPAD, H1), jnp.float32),
            jax.ShapeDtypeStruct((NEXP, H1), jnp.float32),
            jax.ShapeDtypeStruct((NEXP, H1), jnp.float32),
        ],
    )(acc1, p1, dis, b1)


def _tc2b_body(r_ref, dis_ref, w2e_ref, c2_ref, p2_ref):
    for k in range(NEXP):
        r = r_ref[k]
        dis = dis_ref[k, :]
        cols = []
        for c in range(2):
            acc = c2_ref[k, c] + r[:, 0] * w2e_ref[k, 0, c]
            for f in range(1, H1):
                acc = acc + r[:, f] * w2e_ref[k, f, c]
            cols.append(acc * dis)
        cols += [jnp.zeros((NB,), jnp.float32)] * (RW - 2)
        p2_ref[k] = jnp.stack(cols, axis=-1)


def _tc2b(r, dis, W2e, c2):
    return pl.pallas_call(
        _tc2b_body,
        grid=(NBLK,),
        in_specs=[
            pl.BlockSpec((NEXP, NB, H1), lambda i: (0, i, 0)),
            pl.BlockSpec((NEXP, NB), lambda i: (0, i)),
            pl.BlockSpec((NEXP, H1, 2), lambda i: (0, 0, 0)),
            pl.BlockSpec((NEXP, 2), lambda i: (0, 0)),
        ],
        out_specs=pl.BlockSpec((NEXP, NB, RW), lambda i: (0, i, 0)),
        out_shape=jax.ShapeDtypeStruct((NEXP, NPAD, RW), jnp.float32),
    )(r, dis, W2e, c2)


def _tc3_body(acc_ref, p2_ref, dis_ref, b2_ref, fT_ref, wg_ref, bg_ref,
              out_ref):
    fb = fT_ref[...]
    logits = []
    for k in range(NEXP):
        s = bg_ref[k] + wg_ref[0, k] * fb[0, :]
        for j in range(1, FEAT):
            s = s + wg_ref[j, k] * fb[j, :]
        logits.append(s)
    mx = jnp.maximum(jnp.maximum(logits[0], logits[1]),
                     jnp.maximum(logits[2], logits[3]))
    es = [jnp.exp(s - mx) for s in logits]
    den = es[0] + es[1] + es[2] + es[3]
    f0 = jnp.zeros_like(mx)
    f1 = jnp.zeros_like(mx)
    for k in range(NEXP):
        a = acc_ref[0, k, :, :2] + acc_ref[1, k, :, :2]
        o = dis_ref[k, :][:, None] * (a + 2.0 * p2_ref[k, :, :2]) \
            + b2_ref[k, :][None, :]
        o0, o1 = o[:, 0], o[:, 1]
        m = jnp.maximum(o0, o1)
        lse = m + jnp.log(jnp.exp(o0 - m) + jnp.exp(o1 - m))
        gate = es[k] / den
        f0 = f0 + gate * (o0 - lse)
        f1 = f1 + gate * (o1 - lse)
    out_ref[...] = jnp.stack([f0, f1], axis=-1)


def _tc3(acc2, p2, dis, b2, featT, Wg, bg):
    return pl.pallas_call(
        _tc3_body,
        grid=(NBLK,),
        in_specs=[
            pl.BlockSpec((NC, NEXP, NB, RW), lambda i: (0, 0, i, 0)),
            pl.BlockSpec((NEXP, NB, RW), lambda i: (0, i, 0)),
            pl.BlockSpec((NEXP, NB), lambda i: (0, i)),
            pl.BlockSpec((NEXP, 2), lambda i: (0, 0)),
            pl.BlockSpec((FEAT, NB), lambda i: (0, i)),
            pl.BlockSpec((FEAT, NEXP), lambda i: (0, 0)),
            pl.BlockSpec((NEXP,), lambda i: (0,)),
        ],
        out_specs=pl.BlockSpec((NB, 2), lambda i: (i, 0)),
        out_shape=jax.ShapeDtypeStruct((NPAD, 2), jnp.float32),
    )(acc2, p2, dis, b2, featT, Wg, bg)


def kernel(flatten, features, pfcnetworks, mdcbcnetworks, v1cnetworks,
           shanetworks, pfcnetworkweights, mdcbcnetworkweights,
           v1cnetworkweights, shanetworkweights,
           W1, b1, gamma, beta, W2, b2, Wg, bg):
    nets = (pfcnetworks, mdcbcnetworks, v1cnetworks, shanetworks)
    srcs = tuple(n[0, 0] for n in nets)
    dsts = tuple(n[0, 1] for n in nets)

    xT = jnp.pad(flatten[0].T, ((0, 0), (0, NPAD - N_GENES)))
    featT = jnp.pad(features[0].T, ((0, 0), (0, NPAD - N_GENES)))

    ones_src = jnp.zeros((CH_E, RW), jnp.float32).at[:, 0].set(1.0)
    zerosw = jnp.zeros((NT, RW), jnp.float32)

    deg = _sc_deg(*dsts, ones_src, zerosw)
    dis, p1 = _tc1(deg, xT, W1)
    acc1 = _sc_agg(srcs, dsts, p1, zerosw)
    r, s1, s2 = _tc2a(acc1, p1, dis, b1)

    mean = s1 / N_GENES
    var = s2 / N_GENES - mean * mean
    scale = gamma * lax.rsqrt(var + 1e-5)
    shift = beta - mean * scale
    W2e = scale[:, :, None] * W2
    c2 = jnp.einsum("kf,kfc->kc", shift, W2)

    p2 = _tc2b(r, dis, W2e, c2)
    acc2 = _sc_agg(srcs, dsts, p2, zerosw)
    final = _tc3(acc2, p2, dis, b2, featT, Wg, bg)
    return final[:N_GENES]


# speedup vs baseline: 162.1585x; 1.3036x over previous
import functools

import jax
import jax.numpy as jnp
from jax import lax
from jax.experimental import pallas as pl
from jax.experimental.pallas import tpu as pltpu
from jax.experimental.pallas import tpu_sc as plsc

N_GENES = 50000
UNIT = 15
FEAT = 20
H1 = 4
E_EDGES = 3200000
NEXP = 4

NC, NS = 2, 16
NPAD = 50176
NT = NPAD // NS
NB = 512
NBLK = NPAD // NB

RW = 8
E_SC = E_EDGES // NC
CH_E = 2000
NCHUNK = E_SC // CH_E
CPT = (NCHUNK + NS - 1) // NS

@functools.cache
def _mesh():
    return plsc.VectorSubcoreMesh(core_axis_name="c", subcore_axis_name="s",
                                  num_cores=NC, num_subcores=NS)


def _sc_deg(n0, n1, n2, n3, ones_src, zerosw):
    def body(n0r, n1r, n2r, n3r, ones_hbm, zer_hbm, out,
             deg_sh, idx_v, ones_v, sem_ld, sem_sc):
        cid = lax.axis_index("c")
        sid = lax.axis_index("s")
        pltpu.sync_copy(ones_hbm, ones_v)

        for k, nref in enumerate((n0r, n1r, n2r, n3r)):
            sl = pl.ds(sid * NT, NT)
            pltpu.sync_copy(zer_hbm, deg_sh.at[sl, :])
            plsc.subcore_barrier()

            def cbase(j):
                return cid * E_SC + (sid + NS * j) * CH_E

            def valid(j):
                return jnp.logical_and(j >= 0, sid + NS * j < NCHUNK)

            def grp(g, carry):
                for step in range(2):
                    j0 = 2 * g + step
                    j1 = j0 - 1
                    b0 = step
                    b1 = (step + 1) % 2
                    @pl.when(valid(j0))
                    def _():
                        @pl.when(j0 >= 2)
                        def _():
                            pltpu.make_async_copy(
                                ones_v, deg_sh.at[idx_v.at[b0]],
                                sem_sc.at[b0]).wait()
                        pltpu.async_copy(
                            nref.at[0, 1, pl.ds(cbase(j0), CH_E)],
                            idx_v.at[b0], sem_ld.at[b0])
                    @pl.when(valid(j1))
                    def _():
                        pltpu.make_async_copy(
                            nref.at[0, 1, pl.ds(cbase(j1), CH_E)],
                            idx_v.at[b1], sem_ld.at[b1]).wait()
                        pltpu.async_copy(ones_v, deg_sh.at[idx_v.at[b1]],
                                         sem_sc.at[b1], add=True)
                return carry

            lax.fori_loop(0, (CPT + 2) // 2, grp, 0)
            for b in range(2):
                pltpu.make_async_copy(ones_v, deg_sh.at[idx_v.at[b]],
                                      sem_sc.at[b]).wait()
            plsc.subcore_barrier()
            pltpu.sync_copy(deg_sh.at[sl, :], out.at[cid, k, sl, :])
            plsc.subcore_barrier()

    fn = pl.kernel(
        body,
        out_type=jax.ShapeDtypeStruct((NC, NEXP, NPAD, RW), jnp.float32),
        mesh=_mesh(),
        compiler_params=pltpu.CompilerParams(use_tc_tiling_on_sc=False),
        scratch_types=[
            pltpu.VMEM_SHARED((NPAD, RW), jnp.float32),
            pltpu.VMEM((2, CH_E), jnp.int32),
            pltpu.VMEM((CH_E, RW), jnp.float32),
            pltpu.SemaphoreType.DMA((2,)),
            pltpu.SemaphoreType.DMA((2,)),
        ],
    )
    return fn(n0, n1, n2, n3, ones_src, zerosw)


def _sc_agg(nets, p, zerosw):
    def body(n0r, n1r, n2r, n3r, p_hbm, zer_hbm, out,
             p_sh, acc_sh, idx_s, idx_d, rows_v, sem_ld, sem_g, sem_sc):
        cid = lax.axis_index("c")
        sid = lax.axis_index("s")
        for k, nref in enumerate((n0r, n1r, n2r, n3r)):
            sl = pl.ds(sid * NT, NT)
            pltpu.sync_copy(p_hbm.at[k, sl, :], p_sh.at[sl, :])
            pltpu.sync_copy(zer_hbm, acc_sh.at[sl, :])
            plsc.subcore_barrier()

            def cbase(j, row):
                return cid * E_SC + (sid + NS * j) * CH_E

            def valid(j):
                return jnp.logical_and(j >= 0, sid + NS * j < NCHUNK)

            def grp(g, carry):
                for step in range(3):
                    j0 = 3 * g + step
                    j1 = j0 - 1
                    j2 = j0 - 2
                    b0 = step
                    b1 = (step + 2) % 3
                    b2 = (step + 1) % 3
                    @pl.when(valid(j0))
                    def _():
                        @pl.when(j0 >= 3)
                        def _():
                            pltpu.make_async_copy(
                                rows_v.at[b0], acc_sh.at[idx_d.at[b0]],
                                sem_sc.at[b0]).wait()
                        pltpu.async_copy(
                            nref.at[0, 0, pl.ds(cbase(j0, 0), CH_E)],
                            idx_s.at[b0], sem_ld.at[b0])
                        pltpu.async_copy(
                            nref.at[0, 1, pl.ds(cbase(j0, 1), CH_E)],
                            idx_d.at[b0], sem_ld.at[b0])
                    @pl.when(valid(j1))
                    def _():
                        pltpu.make_async_copy(
                            nref.at[0, 0, pl.ds(cbase(j1, 0), CH_E)],
                            idx_s.at[b1], sem_ld.at[b1]).wait()
                        pltpu.make_async_copy(
                            nref.at[0, 1, pl.ds(cbase(j1, 1), CH_E)],
                            idx_d.at[b1], sem_ld.at[b1]).wait()
                        pltpu.async_copy(p_sh.at[idx_s.at[b1]],
                                         rows_v.at[b1], sem_g.at[b1])
                    @pl.when(valid(j2))
                    def _():
                        pltpu.make_async_copy(p_sh.at[idx_s.at[b2]],
                                              rows_v.at[b2],
                                              sem_g.at[b2]).wait()
                        pltpu.async_copy(rows_v.at[b2],
                                         acc_sh.at[idx_d.at[b2]],
                                         sem_sc.at[b2], add=True)
                return carry

            lax.fori_loop(0, (CPT + 4) // 3, grp, 0)
            for b in range(3):
                pltpu.make_async_copy(rows_v.at[b], acc_sh.at[idx_d.at[b]],
                                      sem_sc.at[b]).wait()
            plsc.subcore_barrier()
            pltpu.sync_copy(acc_sh.at[sl, :], out.at[cid, k, sl, :])
            plsc.subcore_barrier()

    fn = pl.kernel(
        body,
        out_type=jax.ShapeDtypeStruct((NC, NEXP, NPAD, RW), jnp.float32),
        mesh=_mesh(),
        compiler_params=pltpu.CompilerParams(use_tc_tiling_on_sc=False),
        scratch_types=[
            pltpu.VMEM_SHARED((NPAD, RW), jnp.float32),
            pltpu.VMEM_SHARED((NPAD, RW), jnp.float32),
            pltpu.VMEM((3, CH_E), jnp.int32),
            pltpu.VMEM((3, CH_E), jnp.int32),
            pltpu.VMEM((3, CH_E, RW), jnp.float32),
            pltpu.SemaphoreType.DMA((3,)),
            pltpu.SemaphoreType.DMA((3,)),
            pltpu.SemaphoreType.DMA((3,)),
        ],
    )
    return fn(*nets, p, zerosw)


def _tc1_body(deg_ref, xT_ref, w1_ref, dis_ref, p1_ref):
    deg = deg_ref[0, :, :, 0] + deg_ref[1, :, :, 0]
    dis = lax.rsqrt(deg + 2.0)
    dis_ref[...] = dis
    xb = xT_ref[...]
    zero = jnp.zeros((NB,), jnp.float32)
    for k in range(NEXP):
        cols = []
        for f in range(H1):
            acc = w1_ref[k, 0, f] * xb[0, :]
            for j in range(1, UNIT):
                acc = acc + w1_ref[k, j, f] * xb[j, :]
            cols.append(acc * dis[k, :])
        cols += [zero] * (RW - H1)
        p1_ref[k, :, :] = jnp.stack(cols, axis=-1)


def _tc1(deg, xT, W1):
    return pl.pallas_call(
        _tc1_body,
        grid=(NBLK,),
        in_specs=[
            pl.BlockSpec((NC, NEXP, NB, RW), lambda i: (0, 0, i, 0)),
            pl.BlockSpec((UNIT, NB), lambda i: (0, i)),
            pl.BlockSpec((NEXP, UNIT, H1), lambda i: (0, 0, 0)),
        ],
        out_specs=[
            pl.BlockSpec((NEXP, NB), lambda i: (0, i)),
            pl.BlockSpec((NEXP, NB, RW), lambda i: (0, i, 0)),
        ],
        out_shape=[
            jax.ShapeDtypeStruct((NEXP, NPAD), jnp.float32),
            jax.ShapeDtypeStruct((NEXP, NPAD, RW), jnp.float32),
        ],
    )(deg, xT, W1)


def _tc2a_body(acc_ref, p1_ref, dis_ref, b1_ref, r_ref, s1_ref, s2_ref):
    i = pl.program_id(0)

    @pl.when(i == 0)
    def _():
        s1_ref[...] = jnp.zeros((NEXP, H1), jnp.float32)
        s2_ref[...] = jnp.zeros((NEXP, H1), jnp.float32)

    node = i * NB + lax.broadcasted_iota(jnp.int32, (NB, H1), 0)
    valid = (node < N_GENES).astype(jnp.float32)
    for k in range(NEXP):
        a = acc_ref[0, k, :, :H1] + acc_ref[1, k, :, :H1]
        p1 = p1_ref[k, :, :H1]
        dis = dis_ref[k, :][:, None]
        out1 = dis * (a + 2.0 * p1) + b1_ref[k, :][None, :]
        r = jnp.maximum(out1, 0.0) * valid
        r_ref[k] = r
        s1_ref[k, :] += jnp.sum(r, axis=0)
        s2_ref[k, :] += jnp.sum(r * r, axis=0)


def _tc2a(acc1, p1, dis, b1):
    return pl.pallas_call(
        _tc2a_body,
        grid=(NBLK,),
        in_specs=[
            pl.BlockSpec((NC, NEXP, NB, RW), lambda i: (0, 0, i, 0)),
            pl.BlockSpec((NEXP, NB, RW), lambda i: (0, i, 0)),
            pl.BlockSpec((NEXP, NB), lambda i: (0, i)),
            pl.BlockSpec((NEXP, H1), lambda i: (0, 0)),
        ],
        out_specs=[
            pl.BlockSpec((NEXP, NB, H1), lambda i: (0, i, 0)),
            pl.BlockSpec((NEXP, H1), lambda i: (0, 0)),
            pl.BlockSpec((NEXP, H1), lambda i: (0, 0)),
        ],
        out_shape=[
            jax.ShapeDtypeStruct((NEXP, NPAD, H1), jnp.float32),
            jax.ShapeDtypeStruct((NEXP, H1), jnp.float32),
            jax.ShapeDtypeStruct((NEXP, H1), jnp.float32),
        ],
    )(acc1, p1, dis, b1)


def _tc2b_body(r_ref, dis_ref, w2e_ref, c2_ref, p2_ref):
    for k in range(NEXP):
        r = r_ref[k]
        dis = dis_ref[k, :]
        cols = []
        for c in range(2):
            acc = c2_ref[k, c] + r[:, 0] * w2e_ref[k, 0, c]
            for f in range(1, H1):
                acc = acc + r[:, f] * w2e_ref[k, f, c]
            cols.append(acc * dis)
        cols += [jnp.zeros((NB,), jnp.float32)] * (RW - 2)
        p2_ref[k] = jnp.stack(cols, axis=-1)


def _tc2b(r, dis, W2e, c2):
    return pl.pallas_call(
        _tc2b_body,
        grid=(NBLK,),
        in_specs=[
            pl.BlockSpec((NEXP, NB, H1), lambda i: (0, i, 0)),
            pl.BlockSpec((NEXP, NB), lambda i: (0, i)),
            pl.BlockSpec((NEXP, H1, 2), lambda i: (0, 0, 0)),
            pl.BlockSpec((NEXP, 2), lambda i: (0, 0)),
        ],
        out_specs=pl.BlockSpec((NEXP, NB, RW), lambda i: (0, i, 0)),
        out_shape=jax.ShapeDtypeStruct((NEXP, NPAD, RW), jnp.float32),
    )(r, dis, W2e, c2)


def _tc3_body(acc_ref, p2_ref, dis_ref, b2_ref, fT_ref, wg_ref, bg_ref,
              out_ref):
    fb = fT_ref[...]
    logits = []
    for k in range(NEXP):
        s = bg_ref[k] + wg_ref[0, k] * fb[0, :]
        for j in range(1, FEAT):
            s = s + wg_ref[j, k] * fb[j, :]
        logits.append(s)
    mx = jnp.maximum(jnp.maximum(logits[0], logits[1]),
                     jnp.maximum(logits[2], logits[3]))
    es = [jnp.exp(s - mx) for s in logits]
    den = es[0] + es[1] + es[2] + es[3]
    f0 = jnp.zeros_like(mx)
    f1 = jnp.zeros_like(mx)
    for k in range(NEXP):
        a = acc_ref[0, k, :, :2] + acc_ref[1, k, :, :2]
        o = dis_ref[k, :][:, None] * (a + 2.0 * p2_ref[k, :, :2]) \
            + b2_ref[k, :][None, :]
        o0, o1 = o[:, 0], o[:, 1]
        m = jnp.maximum(o0, o1)
        lse = m + jnp.log(jnp.exp(o0 - m) + jnp.exp(o1 - m))
        gate = es[k] / den
        f0 = f0 + gate * (o0 - lse)
        f1 = f1 + gate * (o1 - lse)
    out_ref[...] = jnp.stack([f0, f1], axis=-1)


def _tc3(acc2, p2, dis, b2, featT, Wg, bg):
    return pl.pallas_call(
        _tc3_body,
        grid=(NBLK,),
        in_specs=[
            pl.BlockSpec((NC, NEXP, NB, RW), lambda i: (0, 0, i, 0)),
            pl.BlockSpec((NEXP, NB, RW), lambda i: (0, i, 0)),
            pl.BlockSpec((NEXP, NB), lambda i: (0, i)),
            pl.BlockSpec((NEXP, 2), lambda i: (0, 0)),
            pl.BlockSpec((FEAT, NB), lambda i: (0, i)),
            pl.BlockSpec((FEAT, NEXP), lambda i: (0, 0)),
            pl.BlockSpec((NEXP,), lambda i: (0,)),
        ],
        out_specs=pl.BlockSpec((NB, 2), lambda i: (i, 0)),
        out_shape=jax.ShapeDtypeStruct((NPAD, 2), jnp.float32),
    )(acc2, p2, dis, b2, featT, Wg, bg)


def kernel(flatten, features, pfcnetworks, mdcbcnetworks, v1cnetworks,
           shanetworks, pfcnetworkweights, mdcbcnetworkweights,
           v1cnetworkweights, shanetworkweights,
           W1, b1, gamma, beta, W2, b2, Wg, bg):
    nets = (pfcnetworks, mdcbcnetworks, v1cnetworks, shanetworks)

    xT = jnp.pad(flatten[0].T, ((0, 0), (0, NPAD - N_GENES)))
    featT = jnp.pad(features[0].T, ((0, 0), (0, NPAD - N_GENES)))

    ones_src = jnp.zeros((CH_E, RW), jnp.float32).at[:, 0].set(1.0)
    zerosw = jnp.zeros((NT, RW), jnp.float32)

    deg = _sc_deg(*nets, ones_src, zerosw)
    dis, p1 = _tc1(deg, xT, W1)
    acc1 = _sc_agg(nets, p1, zerosw)
    r, s1, s2 = _tc2a(acc1, p1, dis, b1)

    mean = s1 / N_GENES
    var = s2 / N_GENES - mean * mean
    scale = gamma * lax.rsqrt(var + 1e-5)
    shift = beta - mean * scale
    W2e = scale[:, :, None] * W2
    c2 = jnp.einsum("kf,kfc->kc", shift, W2)

    p2 = _tc2b(r, dis, W2e, c2)
    acc2 = _sc_agg(nets, p2, zerosw)
    final = _tc3(acc2, p2, dis, b2, featT, Wg, bg)
    return final[:N_GENES]


# speedup vs baseline: 169.7284x; 1.0467x over previous
import functools

import jax
import jax.numpy as jnp
from jax import lax
from jax.experimental import pallas as pl
from jax.experimental.pallas import tpu as pltpu
from jax.experimental.pallas import tpu_sc as plsc

N_GENES = 50000
UNIT = 15
FEAT = 20
H1 = 4
E_EDGES = 3200000
NEXP = 4

NC, NS = 2, 16
NPAD = 50176
NT = NPAD // NS
NB = 1024
NBLK = NPAD // NB

RW = 8
E_SC = E_EDGES // NC
CH_E = 2000
NCHUNK = E_SC // CH_E
CPT = (NCHUNK + NS - 1) // NS
CH_D = 5000
NCHUNK_D = E_SC // CH_D
CPT_D = (NCHUNK_D + NS - 1) // NS

@functools.cache
def _mesh():
    return plsc.VectorSubcoreMesh(core_axis_name="c", subcore_axis_name="s",
                                  num_cores=NC, num_subcores=NS)


def _sc_deg(n0, n1, n2, n3, ones_src, zerosw):
    def body(n0r, n1r, n2r, n3r, ones_hbm, zer_hbm, out,
             deg_sh, idx_v, ones_v, sem_ld, sem_sc):
        cid = lax.axis_index("c")
        sid = lax.axis_index("s")
        pltpu.sync_copy(ones_hbm, ones_v)

        for k, nref in enumerate((n0r, n1r, n2r, n3r)):
            sl = pl.ds(sid * NT, NT)
            pltpu.sync_copy(zer_hbm, deg_sh.at[sl, :])
            plsc.subcore_barrier()

            def cbase(j):
                return cid * E_SC + (sid + NS * j) * CH_D

            def valid(j):
                return jnp.logical_and(j >= 0, sid + NS * j < NCHUNK_D)

            def grp(g, carry):
                for step in range(2):
                    j0 = 2 * g + step
                    j1 = j0 - 1
                    b0 = step
                    b1 = (step + 1) % 2
                    @pl.when(valid(j0))
                    def _():
                        @pl.when(j0 >= 2)
                        def _():
                            pltpu.make_async_copy(
                                ones_v, deg_sh.at[idx_v.at[b0]],
                                sem_sc.at[b0]).wait()
                        pltpu.async_copy(
                            nref.at[0, 1, pl.ds(cbase(j0), CH_D)],
                            idx_v.at[b0], sem_ld.at[b0])
                    @pl.when(valid(j1))
                    def _():
                        pltpu.make_async_copy(
                            nref.at[0, 1, pl.ds(cbase(j1), CH_D)],
                            idx_v.at[b1], sem_ld.at[b1]).wait()
                        pltpu.async_copy(ones_v, deg_sh.at[idx_v.at[b1]],
                                         sem_sc.at[b1], add=True)
                return carry

            lax.fori_loop(0, (CPT_D + 2) // 2, grp, 0)
            for b in range(2):
                pltpu.make_async_copy(ones_v, deg_sh.at[idx_v.at[b]],
                                      sem_sc.at[b]).wait()
            plsc.subcore_barrier()
            pltpu.sync_copy(deg_sh.at[sl, :], out.at[cid, k, sl, :])
            plsc.subcore_barrier()

    fn = pl.kernel(
        body,
        out_type=jax.ShapeDtypeStruct((NC, NEXP, NPAD, RW), jnp.float32),
        mesh=_mesh(),
        compiler_params=pltpu.CompilerParams(use_tc_tiling_on_sc=False),
        scratch_types=[
            pltpu.VMEM_SHARED((NPAD, RW), jnp.float32),
            pltpu.VMEM((2, CH_D), jnp.int32),
            pltpu.VMEM((CH_D, RW), jnp.float32),
            pltpu.SemaphoreType.DMA((2,)),
            pltpu.SemaphoreType.DMA((2,)),
        ],
    )
    return fn(n0, n1, n2, n3, ones_src, zerosw)


def _sc_agg(nets, p, zerosw):
    def body(n0r, n1r, n2r, n3r, p_hbm, zer_hbm, out,
             p_sh, acc_sh, idx_s, idx_d, rows_v, sem_ld, sem_g, sem_sc):
        cid = lax.axis_index("c")
        sid = lax.axis_index("s")
        for k, nref in enumerate((n0r, n1r, n2r, n3r)):
            sl = pl.ds(sid * NT, NT)
            pltpu.sync_copy(p_hbm.at[k, sl, :], p_sh.at[sl, :])
            pltpu.sync_copy(zer_hbm, acc_sh.at[sl, :])
            plsc.subcore_barrier()

            def cbase(j, row):
                return cid * E_SC + (sid + NS * j) * CH_E

            def valid(j):
                return jnp.logical_and(j >= 0, sid + NS * j < NCHUNK)

            def grp(g, carry):
                for step in range(3):
                    j0 = 3 * g + step
                    j1 = j0 - 1
                    j2 = j0 - 2
                    b0 = step
                    b1 = (step + 2) % 3
                    b2 = (step + 1) % 3
                    @pl.when(valid(j0))
                    def _():
                        @pl.when(j0 >= 3)
                        def _():
                            pltpu.make_async_copy(
                                rows_v.at[b0], acc_sh.at[idx_d.at[b0]],
                                sem_sc.at[b0]).wait()
                        pltpu.async_copy(
                            nref.at[0, 0, pl.ds(cbase(j0, 0), CH_E)],
                            idx_s.at[b0], sem_ld.at[b0])
                        pltpu.async_copy(
                            nref.at[0, 1, pl.ds(cbase(j0, 1), CH_E)],
                            idx_d.at[b0], sem_ld.at[b0])
                    @pl.when(valid(j1))
                    def _():
                        pltpu.make_async_copy(
                            nref.at[0, 0, pl.ds(cbase(j1, 0), CH_E)],
                            idx_s.at[b1], sem_ld.at[b1]).wait()
                        pltpu.make_async_copy(
                            nref.at[0, 1, pl.ds(cbase(j1, 1), CH_E)],
                            idx_d.at[b1], sem_ld.at[b1]).wait()
                        pltpu.async_copy(p_sh.at[idx_s.at[b1]],
                                         rows_v.at[b1], sem_g.at[b1])
                    @pl.when(valid(j2))
                    def _():
                        pltpu.make_async_copy(p_sh.at[idx_s.at[b2]],
                                              rows_v.at[b2],
                                              sem_g.at[b2]).wait()
                        pltpu.async_copy(rows_v.at[b2],
                                         acc_sh.at[idx_d.at[b2]],
                                         sem_sc.at[b2], add=True)
                return carry

            lax.fori_loop(0, (CPT + 4) // 3, grp, 0)
            for b in range(3):
                pltpu.make_async_copy(rows_v.at[b], acc_sh.at[idx_d.at[b]],
                                      sem_sc.at[b]).wait()
            plsc.subcore_barrier()
            pltpu.sync_copy(acc_sh.at[sl, :], out.at[cid, k, sl, :])
            plsc.subcore_barrier()

    fn = pl.kernel(
        body,
        out_type=jax.ShapeDtypeStruct((NC, NEXP, NPAD, RW), jnp.float32),
        mesh=_mesh(),
        compiler_params=pltpu.CompilerParams(use_tc_tiling_on_sc=False),
        scratch_types=[
            pltpu.VMEM_SHARED((NPAD, RW), jnp.float32),
            pltpu.VMEM_SHARED((NPAD, RW), jnp.float32),
            pltpu.VMEM((3, CH_E), jnp.int32),
            pltpu.VMEM((3, CH_E), jnp.int32),
            pltpu.VMEM((3, CH_E, RW), jnp.float32),
            pltpu.SemaphoreType.DMA((3,)),
            pltpu.SemaphoreType.DMA((3,)),
            pltpu.SemaphoreType.DMA((3,)),
        ],
    )
    return fn(*nets, p, zerosw)


def _tc1_body(deg_ref, xT_ref, w1_ref, dis_ref, p1_ref):
    deg = deg_ref[0, :, :, 0] + deg_ref[1, :, :, 0]
    dis = lax.rsqrt(deg + 2.0)
    dis_ref[...] = dis
    xb = xT_ref[...]
    zero = jnp.zeros((NB,), jnp.float32)
    for k in range(NEXP):
        cols = []
        for f in range(H1):
            acc = w1_ref[k, 0, f] * xb[0, :]
            for j in range(1, UNIT):
                acc = acc + w1_ref[k, j, f] * xb[j, :]
            cols.append(acc * dis[k, :])
        cols += [zero] * (RW - H1)
        p1_ref[k, :, :] = jnp.stack(cols, axis=-1)


def _tc1(deg, xT, W1):
    return pl.pallas_call(
        _tc1_body,
        grid=(NBLK,),
        in_specs=[
            pl.BlockSpec((NC, NEXP, NB, RW), lambda i: (0, 0, i, 0)),
            pl.BlockSpec((UNIT, NB), lambda i: (0, i)),
            pl.BlockSpec((NEXP, UNIT, H1), lambda i: (0, 0, 0)),
        ],
        out_specs=[
            pl.BlockSpec((NEXP, NB), lambda i: (0, i)),
            pl.BlockSpec((NEXP, NB, RW), lambda i: (0, i, 0)),
        ],
        out_shape=[
            jax.ShapeDtypeStruct((NEXP, NPAD), jnp.float32),
            jax.ShapeDtypeStruct((NEXP, NPAD, RW), jnp.float32),
        ],
    )(deg, xT, W1)


def _tc2a_body(acc_ref, p1_ref, dis_ref, b1_ref, r_ref, s1_ref, s2_ref):
    i = pl.program_id(0)

    @pl.when(i == 0)
    def _():
        s1_ref[...] = jnp.zeros((NEXP, H1), jnp.float32)
        s2_ref[...] = jnp.zeros((NEXP, H1), jnp.float32)

    node = i * NB + lax.broadcasted_iota(jnp.int32, (NB, H1), 0)
    valid = (node < N_GENES).astype(jnp.float32)
    for k in range(NEXP):
        a = acc_ref[0, k, :, :H1] + acc_ref[1, k, :, :H1]
        p1 = p1_ref[k, :, :H1]
        dis = dis_ref[k, :][:, None]
        out1 = dis * (a + 2.0 * p1) + b1_ref[k, :][None, :]
        r = jnp.maximum(out1, 0.0) * valid
        r_ref[k] = r
        s1_ref[k, :] += jnp.sum(r, axis=0)
        s2_ref[k, :] += jnp.sum(r * r, axis=0)


def _tc2a(acc1, p1, dis, b1):
    return pl.pallas_call(
        _tc2a_body,
        grid=(NBLK,),
        in_specs=[
            pl.BlockSpec((NC, NEXP, NB, RW), lambda i: (0, 0, i, 0)),
            pl.BlockSpec((NEXP, NB, RW), lambda i: (0, i, 0)),
            pl.BlockSpec((NEXP, NB), lambda i: (0, i)),
            pl.BlockSpec((NEXP, H1), lambda i: (0, 0)),
        ],
        out_specs=[
            pl.BlockSpec((NEXP, NB, H1), lambda i: (0, i, 0)),
            pl.BlockSpec((NEXP, H1), lambda i: (0, 0)),
            pl.BlockSpec((NEXP, H1), lambda i: (0, 0)),
        ],
        out_shape=[
            jax.ShapeDtypeStruct((NEXP, NPAD, H1), jnp.float32),
            jax.ShapeDtypeStruct((NEXP, H1), jnp.float32),
            jax.ShapeDtypeStruct((NEXP, H1), jnp.float32),
        ],
    )(acc1, p1, dis, b1)


def _tc2b_body(r_ref, dis_ref, w2e_ref, c2_ref, p2_ref):
    for k in range(NEXP):
        r = r_ref[k]
        dis = dis_ref[k, :]
        cols = []
        for c in range(2):
            acc = c2_ref[k, c] + r[:, 0] * w2e_ref[k, 0, c]
            for f in range(1, H1):
                acc = acc + r[:, f] * w2e_ref[k, f, c]
            cols.append(acc * dis)
        cols += [jnp.zeros((NB,), jnp.float32)] * (RW - 2)
        p2_ref[k] = jnp.stack(cols, axis=-1)


def _tc2b(r, dis, W2e, c2):
    return pl.pallas_call(
        _tc2b_body,
        grid=(NBLK,),
        in_specs=[
            pl.BlockSpec((NEXP, NB, H1), lambda i: (0, i, 0)),
            pl.BlockSpec((NEXP, NB), lambda i: (0, i)),
            pl.BlockSpec((NEXP, H1, 2), lambda i: (0, 0, 0)),
            pl.BlockSpec((NEXP, 2), lambda i: (0, 0)),
        ],
        out_specs=pl.BlockSpec((NEXP, NB, RW), lambda i: (0, i, 0)),
        out_shape=jax.ShapeDtypeStruct((NEXP, NPAD, RW), jnp.float32),
    )(r, dis, W2e, c2)


def _tc3_body(acc_ref, p2_ref, dis_ref, b2_ref, fT_ref, wg_ref, bg_ref,
              out_ref):
    fb = fT_ref[...]
    logits = []
    for k in range(NEXP):
        s = bg_ref[k] + wg_ref[0, k] * fb[0, :]
        for j in range(1, FEAT):
            s = s + wg_ref[j, k] * fb[j, :]
        logits.append(s)
    mx = jnp.maximum(jnp.maximum(logits[0], logits[1]),
                     jnp.maximum(logits[2], logits[3]))
    es = [jnp.exp(s - mx) for s in logits]
    den = es[0] + es[1] + es[2] + es[3]
    f0 = jnp.zeros_like(mx)
    f1 = jnp.zeros_like(mx)
    for k in range(NEXP):
        a = acc_ref[0, k, :, :2] + acc_ref[1, k, :, :2]
        o = dis_ref[k, :][:, None] * (a + 2.0 * p2_ref[k, :, :2]) \
            + b2_ref[k, :][None, :]
        o0, o1 = o[:, 0], o[:, 1]
        m = jnp.maximum(o0, o1)
        lse = m + jnp.log(jnp.exp(o0 - m) + jnp.exp(o1 - m))
        gate = es[k] / den
        f0 = f0 + gate * (o0 - lse)
        f1 = f1 + gate * (o1 - lse)
    out_ref[...] = jnp.stack([f0, f1], axis=-1)


def _tc3(acc2, p2, dis, b2, featT, Wg, bg):
    return pl.pallas_call(
        _tc3_body,
        grid=(NBLK,),
        in_specs=[
            pl.BlockSpec((NC, NEXP, NB, RW), lambda i: (0, 0, i, 0)),
            pl.BlockSpec((NEXP, NB, RW), lambda i: (0, i, 0)),
            pl.BlockSpec((NEXP, NB), lambda i: (0, i)),
            pl.BlockSpec((NEXP, 2), lambda i: (0, 0)),
            pl.BlockSpec((FEAT, NB), lambda i: (0, i)),
            pl.BlockSpec((FEAT, NEXP), lambda i: (0, 0)),
            pl.BlockSpec((NEXP,), lambda i: (0,)),
        ],
        out_specs=pl.BlockSpec((NB, 2), lambda i: (i, 0)),
        out_shape=jax.ShapeDtypeStruct((NPAD, 2), jnp.float32),
    )(acc2, p2, dis, b2, featT, Wg, bg)


def kernel(flatten, features, pfcnetworks, mdcbcnetworks, v1cnetworks,
           shanetworks, pfcnetworkweights, mdcbcnetworkweights,
           v1cnetworkweights, shanetworkweights,
           W1, b1, gamma, beta, W2, b2, Wg, bg):
    nets = (pfcnetworks, mdcbcnetworks, v1cnetworks, shanetworks)

    xT = jnp.pad(flatten[0].T, ((0, 0), (0, NPAD - N_GENES)))
    featT = jnp.pad(features[0].T, ((0, 0), (0, NPAD - N_GENES)))

    ones_src = jnp.zeros((CH_D, RW), jnp.float32).at[:, 0].set(1.0)
    zerosw = jnp.zeros((NT, RW), jnp.float32)

    deg = _sc_deg(*nets, ones_src, zerosw)
    dis, p1 = _tc1(deg, xT, W1)
    acc1 = _sc_agg(nets, p1, zerosw)
    r, s1, s2 = _tc2a(acc1, p1, dis, b1)

    mean = s1 / N_GENES
    var = s2 / N_GENES - mean * mean
    scale = gamma * lax.rsqrt(var + 1e-5)
    shift = beta - mean * scale
    W2e = scale[:, :, None] * W2
    c2 = jnp.einsum("kf,kfc->kc", shift, W2)

    p2 = _tc2b(r, dis, W2e, c2)
    acc2 = _sc_agg(nets, p2, zerosw)
    final = _tc3(acc2, p2, dis, b2, featT, Wg, bg)
    return final[:N_GENES]
